# Initial kernel scaffold; baseline (speedup 1.0000x reference)
import jax
import jax.numpy as jnp
from jax.experimental import pallas as pl


def kernel(x_in, bb1_w, bb1_b, bb2_w, bb2_b, bb3_w, bb3_b, msk1_w, msk1_b, msk2_w, msk2_b, msk3_w, msk3_b, c1a_w, c1a_b, c1b_w, c1b_b, c1c_w, c1c_b, c2a_w, c2a_b, c2b_w, c2b_b, c3a_w, c3a_b, c3b_w, c3b_b, r1_w, r1_b, r2_w, r2_b, r3_w, r3_b):
    raise NotImplementedError("write your pallas kernel here")


# speedup vs baseline: 1.8933x; 1.8933x over previous
import functools

import jax
import jax.numpy as jnp
from jax import lax
from jax.experimental import pallas as pl
from jax.experimental.pallas import tpu as pltpu
from jax.experimental.pallas import tpu_sc as plsc

_N = 50176
_BLK = 1792
_GRID = _N // _BLK

_D = 3200

_NW = 32
_PW = _N // _NW
_C = 16
_ROUNDS = _PW // _C


def _lk(x):
    return jnp.maximum(x, 0.01 * x)


def _argmax2d(x, width):
    m = jnp.max(x, axis=1, keepdims=True)
    io = lax.broadcasted_iota(jnp.int32, x.shape, 1)
    return jnp.min(jnp.where(x == m, io, width), axis=1, keepdims=True)


def _tc_body(xf_ref,
             bb1_w, bb1_b, bb2_w, bb2_b, bb3_w, bb3_b,
             msk1_w, msk1_b, msk2_w, msk2_b, msk3_wT, msk3_b,
             c1a_w, c1a_b, c1b_w, c1b_b, c1c_w, c1c_b,
             c2aT, c2a_b, c2bT, c2b_b,
             r1_w, r1_b, r2T,
             mask_o, l_o, i12_o, r2_o):
    f32 = jnp.float32
    xf = xf_ref[...]
    x = _lk(jnp.dot(xf, bb1_w[...], preferred_element_type=f32) + bb1_b[...])
    x = _lk(jnp.dot(x, bb2_w[...], preferred_element_type=f32) + bb2_b[...])
    x_l = _lk(jnp.dot(x, bb3_w[...], preferred_element_type=f32) + bb3_b[...])
    m = _lk(jnp.dot(xf, msk1_w[...], preferred_element_type=f32) + msk1_b[...])
    m = _lk(jnp.dot(m, msk2_w[...], preferred_element_type=f32) + msk2_b[...])
    mask = _lk(jnp.sum(m * msk3_wT[...], axis=1, keepdims=True) + msk3_b[...])
    mask_o[...] = mask
    l = _lk(jnp.dot(x_l, c1a_w[...], preferred_element_type=f32) + c1a_b[...])
    l = _lk(jnp.dot(l, c1b_w[...], preferred_element_type=f32) + c1b_b[...])
    logits1 = jnp.dot(l, c1c_w[...], preferred_element_type=f32) + c1c_b[...]
    inds1 = _argmax2d(logits1, 16)
    l_o[...] = l
    allo_a = jnp.dot(l, c2aT[...], preferred_element_type=f32)
    h2 = jnp.zeros(allo_a.shape[:1] + (32,), f32)
    for k in range(16):
        mk = inds1 == k
        h2 = jnp.where(mk, allo_a[:, k * 32:(k + 1) * 32] + c2a_b[k:k + 1, :], h2)
    h2 = _lk(h2)
    allo_b = jnp.dot(h2, c2bT[...], preferred_element_type=f32)
    logits2 = jnp.zeros(allo_b.shape[:1] + (32,), f32)
    for k in range(16):
        mk = inds1 == k
        logits2 = jnp.where(mk, allo_b[:, k * 32:(k + 1) * 32] + c2b_b[k:k + 1, :],
                            logits2)
    i2 = _argmax2d(logits2, 32) - 8
    i12 = jnp.clip(inds1 * 16 + i2, 0, 255)
    i12_o[...] = i12
    xr = _lk(jnp.dot(xf, r1_w[...], preferred_element_type=f32) + r1_b[...])
    r2_o[...] = jnp.dot(xr, r2T[...], preferred_element_type=f32)


def _tc_stage(xf, bb1_w, bb1_b, bb2_w, bb2_b, bb3_w, bb3_b,
              msk1_w, msk1_b, msk2_w, msk2_b, msk3_wT, msk3_b,
              c1a_w, c1a_b, c1b_w, c1b_b, c1c_w, c1c_b,
              c2aT, c2a_b, c2bT, c2b_b, r1_w, r1_b, r2T):
    full = lambda a: pl.BlockSpec(a.shape, lambda i: (0,) * a.ndim)
    ins = (bb1_w, bb1_b, bb2_w, bb2_b, bb3_w, bb3_b,
           msk1_w, msk1_b, msk2_w, msk2_b, msk3_wT, msk3_b,
           c1a_w, c1a_b, c1b_w, c1b_b, c1c_w, c1c_b,
           c2aT, c2a_b, c2bT, c2b_b, r1_w, r1_b, r2T)
    return pl.pallas_call(
        _tc_body,
        grid=(_GRID,),
        in_specs=[pl.BlockSpec((_BLK, 128), lambda i: (i, 0))] +
                 [full(a) for a in ins],
        out_specs=[
            pl.BlockSpec((_BLK, 1), lambda i: (i, 0)),
            pl.BlockSpec((_BLK, 32), lambda i: (i, 0)),
            pl.BlockSpec((_BLK, 1), lambda i: (i, 0)),
            pl.BlockSpec((_BLK, 256), lambda i: (i, 0)),
        ],
        out_shape=[
            jax.ShapeDtypeStruct((_N, 1), jnp.float32),
            jax.ShapeDtypeStruct((_N, 32), jnp.float32),
            jax.ShapeDtypeStruct((_N, 1), jnp.int32),
            jax.ShapeDtypeStruct((_N, 256), jnp.float32),
        ],
    )(xf, *ins)


def _sc_body(l_hbm, i12_hbm, r2_hbm, m_hbm, r2b_hbm, out_hbm,
             idx_v, mrows, l_v, r2_v, r2b_v, out_v, sem):
    wid = lax.axis_index("s") * 2 + lax.axis_index("c")
    base_pix = wid * _PW
    pltpu.sync_copy(r2b_hbm, r2b_v)
    io16 = lax.iota(jnp.int32, 16)

    def round_body(r, carry):
        start = base_pix + r * _C
        pltpu.sync_copy(i12_hbm.at[pl.ds(start, _C)], idx_v)
        pltpu.sync_copy(l_hbm.at[pl.ds(start, _C)], l_v)
        pltpu.sync_copy(r2_hbm.at[pl.ds(start, _C)], r2_v)
        pltpu.async_copy(m_hbm.at[idx_v], mrows, sem).wait()
        e_vec = idx_v[...]
        out_acc = jnp.zeros((16,), jnp.float32)

        for p in range(_C):
            lv0 = l_v[p, pl.ds(0, 16)]
            lv1 = l_v[p, pl.ds(16, 16)]
            acc0 = mrows[p, pl.ds(1024, 16)]
            acc1 = mrows[p, pl.ds(1040, 16)]
            for i in range(32):
                li = lv0[i] if i < 16 else lv1[i - 16]
                acc0 = acc0 + li * mrows[p, pl.ds(i * 32, 16)]
                acc1 = acc1 + li * mrows[p, pl.ds(i * 32 + 16, 16)]
            acc0 = _lk(acc0)
            acc1 = _lk(acc1)
            acc2 = mrows[p, pl.ds(2080, 16)]
            acc3 = mrows[p, pl.ds(2096, 16)]
            for i in range(32):
                hi = acc0[i] if i < 16 else acc1[i - 16]
                acc2 = acc2 + hi * mrows[p, pl.ds(1056 + i * 32, 16)]
                acc3 = acc3 + hi * mrows[p, pl.ds(1056 + i * 32 + 16, 16)]
            mx = jnp.maximum(jnp.max(acc2), jnp.max(acc3))
            id0 = jnp.min(jnp.where(acc2 == mx, io16, 64))
            id1 = jnp.min(jnp.where(acc3 == mx, io16 + 16, 64))
            am = jnp.minimum(id0, id1)
            e = e_vec[p]
            inds_p = jnp.clip(e * 16 + (am - 8), 0, 4095)
            off = inds_p - jnp.clip(e * 16 - 8, 0, 4064)
            sup = inds_p // 512
            xr0 = _lk(r2_v[p, pl.ds(sup * 32, 16)] + r2b_v[sup, pl.ds(0, 16)])
            xr1 = _lk(r2_v[p, pl.ds(sup * 32 + 16, 16)] + r2b_v[sup, pl.ds(16, 16)])
            rw0 = mrows[p, pl.ds(2112 + off * 32, 16)]
            rw1 = mrows[p, pl.ds(2112 + off * 32 + 16, 16)]
            rb0 = mrows[p, pl.ds(3136, 16)]
            rb1 = mrows[p, pl.ds(3152, 16)]
            rb = jnp.sum(jnp.where(io16 == off, rb0, 0.0) +
                         jnp.where(io16 == off - 16, rb1, 0.0))
            rr = jnp.sum(xr0 * rw0 + xr1 * rw1) + rb
            val = (inds_p.astype(jnp.float32) + rr) * (1.0 / 4096.0)
            out_acc = jnp.where(io16 == p, val, out_acc)

        out_v[...] = out_acc
        pltpu.sync_copy(out_v, out_hbm.at[pl.ds(start, _C)])
        return carry

    lax.fori_loop(0, _ROUNDS, round_body, 0)


def _sc_stage(l, i12, r2all, mtab, r2_b):
    mesh = plsc.VectorSubcoreMesh(core_axis_name="c", subcore_axis_name="s")
    f = functools.partial(
        pl.kernel,
        out_type=jax.ShapeDtypeStruct((_N,), jnp.float32),
        mesh=mesh,
        scratch_types=[
            pltpu.VMEM((_C,), jnp.int32),
            pltpu.VMEM((_C, _D), jnp.float32),
            pltpu.VMEM((_C, 32), jnp.float32),
            pltpu.VMEM((_C, 256), jnp.float32),
            pltpu.VMEM((8, 32), jnp.float32),
            pltpu.VMEM((_C,), jnp.float32),
            pltpu.SemaphoreType.DMA,
        ],
        compiler_params=pltpu.CompilerParams(needs_layout_passes=False),
    )(_sc_body)
    return f(l, i12, r2all, mtab, r2_b)


def kernel(x_in, bb1_w, bb1_b, bb2_w, bb2_b, bb3_w, bb3_b,
           msk1_w, msk1_b, msk2_w, msk2_b, msk3_w, msk3_b,
           c1a_w, c1a_b, c1b_w, c1b_b, c1c_w, c1c_b,
           c2a_w, c2a_b, c2b_w, c2b_b, c3a_w, c3a_b, c3b_w, c3b_b,
           r1_w, r1_b, r2_w, r2_b, r3_w, r3_b):
    b, ch, hh, ww = x_in.shape
    xf = jnp.transpose(x_in, (0, 2, 3, 1)).reshape(_N, ch)
    c2aT = jnp.transpose(c2a_w, (1, 0, 2)).reshape(32, 512)
    c2bT = jnp.transpose(c2b_w, (1, 0, 2)).reshape(32, 512)
    r2T = jnp.transpose(r2_w, (1, 0, 2)).reshape(128, 256)
    msk3_wT = msk3_w.reshape(1, 16)
    j = jnp.arange(256, dtype=jnp.int32)
    rbase = jnp.clip(j * 16 - 8, 0, 4096 - 32)
    rows = rbase[:, None] + jnp.arange(32, dtype=jnp.int32)[None, :]
    r3flat = r3_w[:, :, 0]
    rblk = r3flat[rows].reshape(256, 1024)
    rbblk = r3_b[:, 0][rows]
    mtab = jnp.concatenate(
        [c3a_w.reshape(256, 1024), c3a_b,
         c3b_w.reshape(256, 1024), c3b_b,
         rblk, rbblk,
         jnp.zeros((256, 32), jnp.float32)], axis=1)

    mask_f, l, i12, r2all = _tc_stage(
        xf, bb1_w, bb1_b.reshape(1, 128), bb2_w, bb2_b.reshape(1, 128),
        bb3_w, bb3_b.reshape(1, 128),
        msk1_w, msk1_b.reshape(1, 32), msk2_w, msk2_b.reshape(1, 16),
        msk3_wT, msk3_b.reshape(1, 1),
        c1a_w, c1a_b.reshape(1, 32), c1b_w, c1b_b.reshape(1, 32),
        c1c_w, c1c_b.reshape(1, 16),
        c2aT, c2a_b, c2bT, c2b_b,
        r1_w, r1_b.reshape(1, 128), r2T)

    out_flat = _sc_stage(l, i12.reshape(_N), r2all, mtab, r2_b)
    out = out_flat.reshape(b, 1, hh, ww)
    mask = mask_f.reshape(b, 1, hh, ww)
    return out, mask
